# baseline (speedup 1.0000x reference)
import functools

import jax
import jax.numpy as jnp
from jax import lax
from jax.experimental import pallas as pl
from jax.experimental.pallas import tpu as pltpu
from jax.experimental.pallas import tpu_sc as plsc

_NUM_CORES = 2
_NUM_SUBCORES = 16
_NW = _NUM_CORES * _NUM_SUBCORES
_LANES = 16


def _make_pool(ctx, b, d, bh, hoff):
    rows_per_w = bh // _NW
    n_cvec = d // _LANES
    mesh = plsc.VectorSubcoreMesh(core_axis_name="c", subcore_axis_name="s")

    chunk = 32
    n_chunks = rows_per_w // chunk

    @functools.partial(
        pl.kernel,
        out_type=jax.ShapeDtypeStruct((bh * d,), jnp.float32),
        mesh=mesh,
        scratch_types=[
            pltpu.VMEM((ctx, chunk), jnp.int32),
            pltpu.VMEM((ctx, chunk), jnp.int32),
            pltpu.VMEM((ctx, chunk, d), jnp.float32),
            pltpu.VMEM((ctx, chunk, d), jnp.float32),
            pltpu.VMEM((rows_per_w * d,), jnp.float32),
            pltpu.SemaphoreType.DMA,
            pltpu.SemaphoreType.DMA,
        ],
        compiler_params=pltpu.CompilerParams(use_tc_tiling_on_sc=False),
    )
    def pool(xf_hbm, table_hbm, h_hbm, idx0, idx1, rows0, rows1, acc_v, s0, s1):
        wid = lax.axis_index("s") * _NUM_CORES + lax.axis_index("c")
        lbase = wid * rows_per_w
        base = hoff + lbase
        idx = (idx0, idx1)
        rows = (rows0, rows1)
        sems = (s0, s1)

        def fire(k):
            p = k % 2
            handles = []
            for j in range(ctx):
                pltpu.sync_copy(
                    xf_hbm.at[pl.ds(j * b + base + k * chunk, chunk)],
                    idx[p].at[j],
                )
                handles.append(
                    pltpu.async_copy(
                        table_hbm.at[idx[p].at[j]], rows[p].at[j], sems[p]
                    )
                )
            return handles

        handles = [fire(0), None]
        for k in range(n_chunks):
            if k + 1 < n_chunks:
                handles[(k + 1) % 2] = fire(k + 1)
            for hd in handles[k % 2]:
                hd.wait()
            rv = rows[k % 2]

            def reduce_row(r, carry):
                for c in range(n_cvec):
                    sl = pl.ds(c * _LANES, _LANES)
                    v = rv[0, r, sl]
                    for j in range(1, ctx):
                        v = v + rv[j, r, sl]
                    acc_v[pl.ds((k * chunk + r) * d + c * _LANES, _LANES)] = v
                return carry

            lax.fori_loop(0, chunk, reduce_row, 0)

        pltpu.sync_copy(acc_v, h_hbm.at[pl.ds(lbase * d, rows_per_w * d)])

    return pool


def _matmul_body(scale, h_ref, wt_ref, b_ref, out_ref):
    h = h_ref[...] * scale
    acc = lax.dot_general(
        wt_ref[...],
        h,
        dimension_numbers=(((0,), (1,)), ((), ())),
        preferred_element_type=jnp.float32,
    )
    out_ref[...] = acc + jnp.transpose(b_ref[...])


def _projection(h_sum, wt, b2d, ctx, vb):
    batch, d = h_sum.shape
    vocab = wt.shape[1]
    nv = pl.cdiv(vocab, vb)
    out_t = pl.pallas_call(
        functools.partial(_matmul_body, float(1.0 / ctx)),
        grid=(nv,),
        in_specs=[
            pl.BlockSpec((batch, d), lambda j: (0, 0)),
            pl.BlockSpec((d, vb), lambda j: (0, j)),
            pl.BlockSpec((1, vb), lambda j: (0, j)),
        ],
        out_specs=pl.BlockSpec((vb, batch), lambda j: (j, 0)),
        out_shape=jax.ShapeDtypeStruct((vocab, batch), jnp.float32),
        compiler_params=pltpu.CompilerParams(
            dimension_semantics=("arbitrary",),
        ),
    )(h_sum, wt, b2d)
    return out_t.T


def kernel(x, emb_table, W, b):
    batch, ctx = x.shape
    vocab, d = W.shape
    x_flat = x.T.reshape(-1)
    h_sum = _make_pool(ctx, batch, d, batch, 0)(x_flat, emb_table).reshape(
        batch, d
    )
    return _projection(h_sum, W.T, b.reshape(1, vocab), ctx, 1024)


# speedup vs baseline: 1.0318x; 1.0318x over previous
import functools

import jax
import jax.numpy as jnp
from jax import lax
from jax.experimental import pallas as pl
from jax.experimental.pallas import tpu as pltpu
from jax.experimental.pallas import tpu_sc as plsc

_NUM_CORES = 2
_NUM_SUBCORES = 16
_NW = _NUM_CORES * _NUM_SUBCORES
_LANES = 16


def _make_pool(ctx, b, d, bh, hoff):
    rows_per_w = bh // _NW
    n_cvec = d // _LANES
    mesh = plsc.VectorSubcoreMesh(core_axis_name="c", subcore_axis_name="s")

    @functools.partial(
        pl.kernel,
        out_type=jax.ShapeDtypeStruct((bh * d,), jnp.float32),
        mesh=mesh,
        scratch_types=[
            pltpu.VMEM((rows_per_w,), jnp.int32),
            pltpu.VMEM((rows_per_w,), jnp.int32),
            pltpu.VMEM((rows_per_w, d), jnp.float32),
            pltpu.VMEM((rows_per_w, d), jnp.float32),
            pltpu.VMEM((rows_per_w * d,), jnp.float32),
            pltpu.SemaphoreType.DMA,
            pltpu.SemaphoreType.DMA,
        ],
        compiler_params=pltpu.CompilerParams(use_tc_tiling_on_sc=False),
    )
    def pool(xf_hbm, table_hbm, h_hbm, idx0, idx1, rows0, rows1, acc_v, s0, s1):
        wid = lax.axis_index("s") * _NUM_CORES + lax.axis_index("c")
        lbase = wid * rows_per_w
        base = hoff + lbase
        idx = (idx0, idx1)
        rows = (rows0, rows1)
        sems = (s0, s1)

        def start(j):
            k = j % 2
            pltpu.sync_copy(xf_hbm.at[pl.ds(j * b + base, rows_per_w)], idx[k])
            return pltpu.async_copy(table_hbm.at[idx[k]], rows[k], sems[k])

        handles = [start(0), start(1)]
        for j in range(ctx):
            handles[j % 2].wait()
            rv = rows[j % 2]

            if j == 0:

                def init_row(r, carry):
                    for c in range(n_cvec):
                        acc_v[pl.ds(r * d + c * _LANES, _LANES)] = rv[
                            r, pl.ds(c * _LANES, _LANES)
                        ]
                    return carry

                lax.fori_loop(0, rows_per_w, init_row, 0)
            else:

                def add_row(r, carry):
                    for c in range(n_cvec):
                        acc_v[pl.ds(r * d + c * _LANES, _LANES)] += rv[
                            r, pl.ds(c * _LANES, _LANES)
                        ]
                    return carry

                lax.fori_loop(0, rows_per_w, add_row, 0)

            if j + 2 < ctx:
                handles[j % 2] = start(j + 2)

        pltpu.sync_copy(acc_v, h_hbm.at[pl.ds(lbase * d, rows_per_w * d)])

    return pool


def _matmul_body(scale, h_ref, wt_ref, b_ref, out_ref):
    h = h_ref[...] * scale
    acc = lax.dot_general(
        wt_ref[...],
        h,
        dimension_numbers=(((0,), (1,)), ((), ())),
        preferred_element_type=jnp.float32,
    )
    out_ref[...] = acc + jnp.transpose(b_ref[...])


def _projection(h_sum, wt, b2d, ctx, vb):
    batch, d = h_sum.shape
    vocab = wt.shape[1]
    nv = pl.cdiv(vocab, vb)
    out_t = pl.pallas_call(
        functools.partial(_matmul_body, float(1.0 / ctx)),
        grid=(nv,),
        in_specs=[
            pl.BlockSpec((batch, d), lambda j: (0, 0)),
            pl.BlockSpec((d, vb), lambda j: (0, j)),
            pl.BlockSpec((1, vb), lambda j: (0, j)),
        ],
        out_specs=pl.BlockSpec((vb, batch), lambda j: (j, 0)),
        out_shape=jax.ShapeDtypeStruct((vocab, batch), jnp.float32),
        compiler_params=pltpu.CompilerParams(
            dimension_semantics=("arbitrary",),
        ),
    )(h_sum, wt, b2d)
    return out_t.T


def kernel(x, emb_table, W, b):
    batch, ctx = x.shape
    vocab, d = W.shape
    x_flat = x.T.reshape(-1)
    h_sum = _make_pool(ctx, batch, d, batch, 0)(x_flat, emb_table).reshape(
        batch, d
    )
    return _projection(h_sum, W.T, b.reshape(1, vocab), ctx, 1024)
